# Initial kernel scaffold; baseline (speedup 1.0000x reference)
import jax
import jax.numpy as jnp
from jax.experimental import pallas as pl


def kernel(x_host, x_user, edge_index_host_auth_host, edge_index_user_authenticates_to_host, proj_host_w1, proj_host_b1, proj_user_w1, proj_user_b1, att_src_hh1, att_dst_hh1, att_src_uh1, att_dst_uh1, k_lin_w1, k_lin_b1, q1, proj_host_w2, proj_host_b2, proj_user_w2, proj_user_b2, att_src_hh2, att_dst_hh2, att_src_uh2, att_dst_uh2, k_lin_w2, k_lin_b2, q2, proj_w, proj_b):
    raise NotImplementedError("write your pallas kernel here")


# speedup vs baseline: 1.0630x; 1.0630x over previous
import jax
import jax.numpy as jnp
from jax.experimental import pallas as pl


def _proj_kernel(x_ref, w_ref, b_ref, o_ref):
    o_ref[...] = (
        jnp.dot(x_ref[...], w_ref[...], preferred_element_type=jnp.float32)
        + b_ref[...]
    )


def _proj(x, w, b, blk=5000):
    n, k = x.shape
    m = w.shape[1]
    return pl.pallas_call(
        _proj_kernel,
        grid=(n // blk,),
        in_specs=[
            pl.BlockSpec((blk, k), lambda i: (i, 0)),
            pl.BlockSpec((k, m), lambda i: (0, 0)),
            pl.BlockSpec((1, m), lambda i: (0, 0)),
        ],
        out_specs=pl.BlockSpec((blk, m), lambda i: (i, 0)),
        out_shape=jax.ShapeDtypeStruct((n, m), jnp.float32),
    )(x, w, b.reshape(1, m))


def _logit_kernel(s_ref, d_ref, o_ref):
    al = s_ref[...] + d_ref[...]
    o_ref[...] = jnp.where(al >= 0.0, al, 0.2 * al)


def _edge_logits(als_g, ald_g, blk=8000):
    e, h = als_g.shape
    return pl.pallas_call(
        _logit_kernel,
        grid=(e // blk,),
        in_specs=[
            pl.BlockSpec((blk, h), lambda i: (i, 0)),
            pl.BlockSpec((blk, h), lambda i: (i, 0)),
        ],
        out_specs=pl.BlockSpec((blk, h), lambda i: (i, 0)),
        out_shape=jax.ShapeDtypeStruct((e, h), jnp.float32),
    )(als_g, ald_g)


def _msg_kernel(al_ref, m_ref, x_ref, e_ref, msg_ref):
    ea = jnp.exp(al_ref[...] - m_ref[...])
    e_ref[...] = ea
    msg_ref[...] = x_ref[...] * ea[:, :, None]


def _edge_msgs(al, m_g, x_g, blk=1000):
    e, h = al.shape
    d = x_g.shape[2]
    return pl.pallas_call(
        _msg_kernel,
        grid=(e // blk,),
        in_specs=[
            pl.BlockSpec((blk, h), lambda i: (i, 0)),
            pl.BlockSpec((blk, h), lambda i: (i, 0)),
            pl.BlockSpec((blk, h, d), lambda i: (i, 0, 0)),
        ],
        out_specs=[
            pl.BlockSpec((blk, h), lambda i: (i, 0)),
            pl.BlockSpec((blk, h, d), lambda i: (i, 0, 0)),
        ],
        out_shape=[
            jax.ShapeDtypeStruct((e, h), jnp.float32),
            jax.ShapeDtypeStruct((e, h, d), jnp.float32),
        ],
    )(al, m_g, x_g)


def _agg(x_src, x_dst, ei, a_s, a_d, heads, D, n_dst):
    src, dst = ei[0], ei[1]
    al_s = (x_src * a_s[None, :, :]).sum(-1)
    al_d = (x_dst * a_d[None, :, :]).sum(-1)
    al = _edge_logits(al_s[src], al_d[dst])
    m = jax.ops.segment_max(al, dst, num_segments=n_dst)
    m = jnp.where(jnp.isfinite(m), m, 0.0)
    ea, msg = _edge_msgs(al, m[dst], x_src[src])
    ssum = jax.ops.segment_sum(ea, dst, num_segments=n_dst)
    out = jax.ops.segment_sum(msg, dst, num_segments=n_dst)
    out = out / (ssum[:, :, None] + 1e-16)
    return out.reshape(n_dst, heads * D)


def _han(x_h, x_u, ei_hh, ei_uh, ph_w, ph_b, pu_w, pu_b, as_hh, ad_hh,
         as_uh, ad_uh, k_w, k_b, q, heads, out_ch):
    D = out_ch // heads
    xh = _proj(x_h, ph_w, ph_b).reshape(-1, heads, D)
    xu = _proj(x_u, pu_w, pu_b).reshape(-1, heads, D)
    n = xh.shape[0]
    o_hh = jax.nn.relu(_agg(xh, xh, ei_hh, as_hh, ad_hh, heads, D, n))
    o_uh = jax.nn.relu(_agg(xu, xh, ei_uh, as_uh, ad_uh, heads, D, n))
    stk = jnp.stack([o_hh, o_uh])
    kmat = jnp.tanh(stk @ k_w + k_b).mean(axis=1)
    score = (q[None, :] * kmat).sum(-1)
    attn = jax.nn.softmax(score)
    return (attn[:, None, None] * stk).sum(0)


def kernel(x_host, x_user, edge_index_host_auth_host,
           edge_index_user_authenticates_to_host,
           proj_host_w1, proj_host_b1, proj_user_w1, proj_user_b1,
           att_src_hh1, att_dst_hh1, att_src_uh1, att_dst_uh1,
           k_lin_w1, k_lin_b1, q1,
           proj_host_w2, proj_host_b2, proj_user_w2, proj_user_b2,
           att_src_hh2, att_dst_hh2, att_src_uh2, att_dst_uh2,
           k_lin_w2, k_lin_b2, q2, proj_w, proj_b):
    ei_hh = edge_index_host_auth_host
    ei_uh = edge_index_user_authenticates_to_host
    h_host = jax.nn.relu(_han(
        x_host, x_user, ei_hh, ei_uh, proj_host_w1, proj_host_b1,
        proj_user_w1, proj_user_b1, att_src_hh1, att_dst_hh1,
        att_src_uh1, att_dst_uh1, k_lin_w1, k_lin_b1, q1, 4, 64))
    h_user = jnp.zeros((x_user.shape[0], 64), dtype=jnp.float32)
    o_host = _han(
        h_host, h_user, ei_hh, ei_uh, proj_host_w2, proj_host_b2,
        proj_user_w2, proj_user_b2, att_src_hh2, att_dst_hh2,
        att_src_uh2, att_dst_uh2, k_lin_w2, k_lin_b2, q2, 1, 64)
    emb = o_host.mean(axis=0)
    return emb @ proj_w + proj_b
